# Initial kernel scaffold; baseline (speedup 1.0000x reference)
import jax
import jax.numpy as jnp
from jax.experimental import pallas as pl


def kernel(x, edge_index, pos_edge_index, neg_edge_index, Wl1, bl1, Wr1, Wl2, bl2, Wr2):
    raise NotImplementedError("write your pallas kernel here")


# speedup vs baseline: 2.8015x; 2.8015x over previous
import jax
import jax.numpy as jnp
from jax import lax
from jax.experimental import pallas as pl
from jax.experimental.pallas import tpu as pltpu
from jax.experimental.pallas import tpu_sc as plsc

N = 10000
E = 320000
P = 100000
D_IN = 128
D_H = 128
D_OUT = 64

NC = 2
NS = 16
NW = NC * NS

NP = 10240
ROWS_PER_SUB = NP // NS
ZFILL = 40

NCH_E = 200
CE = 50

PP = 102400
NCH_D = 160
CD = 40

_HIGH = lax.Precision.HIGHEST

_sc_mesh = plsc.VectorSubcoreMesh(
    core_axis_name="c", subcore_axis_name="s", num_cores=NC, num_subcores=NS
)


def _dot(a, b):
    return lax.dot_general(a, b, (((1,), (0,)), ((), ())), precision=_HIGH)


def _mm_body(x_ref, w_ref, o_ref):
    o_ref[...] = _dot(x_ref[...], w_ref[...])


def _matmul(x, w, block_rows=2000):
    n, k = x.shape
    m = w.shape[1]
    return pl.pallas_call(
        _mm_body,
        grid=(n // block_rows,),
        in_specs=[
            pl.BlockSpec((block_rows, k), lambda i: (i, 0)),
            pl.BlockSpec((k, m), lambda i: (0, 0)),
        ],
        out_specs=pl.BlockSpec((block_rows, m), lambda i: (i, 0)),
        out_shape=jax.ShapeDtypeStruct((n, m), x.dtype),
    )(x, w)


def _l1_post_body(s1p_ref, cntp_ref, xr_ref, bl1_ref, h_ref, invc_ref):
    s = s1p_ref[0] + s1p_ref[1]
    cnt = cntp_ref[0, :, 0:1] + cntp_ref[1, :, 0:1]
    inv = 1.0 / jnp.maximum(cnt, 1.0)
    h_ref[...] = jnp.maximum(s * inv + bl1_ref[...] + xr_ref[...], 0.0)
    invc_ref[...] = jnp.broadcast_to(inv, invc_ref.shape)


def _l1_post(s1p, cntp, xr, bl1, block_rows=2000):
    return pl.pallas_call(
        _l1_post_body,
        grid=(N // block_rows,),
        in_specs=[
            pl.BlockSpec((2, block_rows, D_H), lambda i: (0, i, 0)),
            pl.BlockSpec((2, block_rows, D_H), lambda i: (0, i, 0)),
            pl.BlockSpec((block_rows, D_H), lambda i: (i, 0)),
            pl.BlockSpec((1, D_H), lambda i: (0, 0)),
        ],
        out_specs=[
            pl.BlockSpec((block_rows, D_H), lambda i: (i, 0)),
            pl.BlockSpec((block_rows, 16), lambda i: (i, 0)),
        ],
        out_shape=[
            jax.ShapeDtypeStruct((N, D_H), jnp.float32),
            jax.ShapeDtypeStruct((N, 16), jnp.float32),
        ],
    )(s1p, cntp, xr, bl1)


def _l2_post_body(s2p_ref, invc_ref, h_ref, wl2_ref, wr2_ref, bl2_ref, z_ref):
    mean = (s2p_ref[0] + s2p_ref[1]) * invc_ref[:, 0:1]
    z64 = _dot(mean, wl2_ref[...]) + bl2_ref[...] + _dot(
        h_ref[...], wr2_ref[...])
    z_ref[...] = jnp.concatenate(
        [z64, jnp.zeros_like(z64)], axis=1)


def _l2_post(s2p, invc, h, wl2, wr2, bl2, block_rows=2000):
    return pl.pallas_call(
        _l2_post_body,
        grid=(N // block_rows,),
        in_specs=[
            pl.BlockSpec((2, block_rows, D_H), lambda i: (0, i, 0)),
            pl.BlockSpec((block_rows, 16), lambda i: (i, 0)),
            pl.BlockSpec((block_rows, D_H), lambda i: (i, 0)),
            pl.BlockSpec((D_H, D_OUT), lambda i: (0, 0)),
            pl.BlockSpec((D_H, D_OUT), lambda i: (0, 0)),
            pl.BlockSpec((1, D_OUT), lambda i: (0, 0)),
        ],
        out_specs=pl.BlockSpec((block_rows, D_H), lambda i: (i, 0)),
        out_shape=jax.ShapeDtypeStruct((N, D_H), jnp.float32),
    )(s2p, invc, h, wl2, wr2, bl2)


def _rowdot_body(a_ref, b_ref, o_ref):
    o_ref[...] = jnp.sum(a_ref[...] * b_ref[...], axis=1, keepdims=True)


def _rowdot(a, b, block_rows=2048):
    n = a.shape[0]
    return pl.pallas_call(
        _rowdot_body,
        grid=(n // block_rows,),
        in_specs=[
            pl.BlockSpec((block_rows, D_H), lambda i: (i, 0)),
            pl.BlockSpec((block_rows, D_H), lambda i: (i, 0)),
        ],
        out_specs=pl.BlockSpec((block_rows, 1), lambda i: (i, 0)),
        out_shape=jax.ShapeDtypeStruct((n, 1), jnp.float32),
    )(a, b)


def _zero_vmem(ref, rows, width):
    zero = jnp.zeros((16,), jnp.float32)

    @pl.loop(0, rows)
    def _(r):
        @pl.loop(0, width, step=16)
        def _(c):
            ref[r, pl.ds(c, 16)] = zero


def _one_vmem(ref, rows, width):
    one = jnp.ones((16,), jnp.float32)

    @pl.loop(0, rows)
    def _(r):
        @pl.loop(0, width, step=16)
        def _(c):
            ref[r, pl.ds(c, 16)] = one


def _fill_stripe(acc, zbuf):
    sid = lax.axis_index("s")
    base = sid * ROWS_PER_SUB
    for k in range(ROWS_PER_SUB // ZFILL):
        pltpu.sync_copy(zbuf.at[pl.ds(0, ZFILL)],
                        acc.at[pl.ds(base + k * ZFILL, ZFILL)])


def _stage_table(tab_h, tab_sh):
    sid = lax.axis_index("s")

    @pl.when(sid < NS - 1)
    def _():
        sl = pl.ds(sid * ROWS_PER_SUB, ROWS_PER_SUB)
        pltpu.sync_copy(tab_h.at[sl], tab_sh.at[sl])

    @pl.when(sid == NS - 1)
    def _():
        last = (NS - 1) * ROWS_PER_SUB
        sl = pl.ds(last, N - last)
        pltpu.sync_copy(tab_h.at[sl], tab_sh.at[sl])


def _segsum128_body(table_h, edges_h, out_h, acc, src2, dst2, rows0, rows1,
                    sem0, sem1):
    src_h = edges_h.at[0]
    dst_h = edges_h.at[1]
    cid = lax.axis_index("c")
    sid = lax.axis_index("s")
    wid = cid * NS + sid

    _zero_vmem(rows0, ZFILL, D_H)
    _fill_stripe(acc, rows0)
    plsc.subcore_barrier()

    @pl.loop(0, NCH_E // 2)
    def _(j):
        pltpu.sync_copy(src_h.at[wid].at[j], src2)
        pltpu.sync_copy(dst_h.at[wid].at[j], dst2)
        g0 = pltpu.async_copy(table_h.at[src2.at[0]], rows0, sem0)
        g1 = pltpu.async_copy(table_h.at[src2.at[1]], rows1, sem1)
        g0.wait()
        pltpu.sync_copy(rows0, acc.at[dst2.at[0]], add=True)
        g1.wait()
        pltpu.sync_copy(rows1, acc.at[dst2.at[1]], add=True)

    plsc.subcore_barrier()
    stripe = pl.ds(sid * ROWS_PER_SUB, ROWS_PER_SUB)
    pltpu.sync_copy(acc.at[stripe], out_h.at[cid].at[stripe])


_segsum128 = pl.kernel(
    _segsum128_body,
    out_type=jax.ShapeDtypeStruct((NC, NP, D_H), jnp.float32),
    mesh=_sc_mesh,
    scratch_types=[
        pltpu.VMEM_SHARED((NP, D_H), jnp.float32),
        pltpu.VMEM((2, CE), jnp.int32),
        pltpu.VMEM((2, CE), jnp.int32),
        pltpu.VMEM((CE, D_H), jnp.float32),
        pltpu.VMEM((CE, D_H), jnp.float32),
        pltpu.SemaphoreType.DMA,
        pltpu.SemaphoreType.DMA,
    ],
)


def _count_body(edges_h, cnt_h, acc, dst2, ones_v):
    dst_h = edges_h.at[1]
    cid = lax.axis_index("c")
    sid = lax.axis_index("s")
    wid = cid * NS + sid

    _zero_vmem(ones_v, ZFILL, D_H)
    _fill_stripe(acc, ones_v)
    _one_vmem(ones_v, CE, D_H)
    plsc.subcore_barrier()

    @pl.loop(0, NCH_E // 2)
    def _(j):
        pltpu.sync_copy(dst_h.at[wid].at[j], dst2)
        pltpu.sync_copy(ones_v, acc.at[dst2.at[0]], add=True)
        pltpu.sync_copy(ones_v, acc.at[dst2.at[1]], add=True)

    plsc.subcore_barrier()
    stripe = pl.ds(sid * ROWS_PER_SUB, ROWS_PER_SUB)
    pltpu.sync_copy(acc.at[stripe], cnt_h.at[cid].at[stripe])


_count_edges = pl.kernel(
    _count_body,
    out_type=jax.ShapeDtypeStruct((NC, NP, D_H), jnp.float32),
    mesh=_sc_mesh,
    scratch_types=[
        pltpu.VMEM_SHARED((NP, D_H), jnp.float32),
        pltpu.VMEM((2, CE), jnp.int32),
        pltpu.VMEM((CE, D_H), jnp.float32),
    ],
)


def _decode_gather_body(z_h, ab_h, zab_h, ai2, bi2, ra0, rb0, ra1, rb1,
                        sa0, sb0, sa1, sb1):
    ai_h = ab_h.at[0]
    bi_h = ab_h.at[1]
    za_h = zab_h.at[0]
    zb_h = zab_h.at[1]
    cid = lax.axis_index("c")
    sid = lax.axis_index("s")
    wid = cid * NS + sid

    @pl.loop(0, NCH_D // 2)
    def _(j):
        pltpu.sync_copy(ai_h.at[wid].at[j], ai2)
        pltpu.sync_copy(bi_h.at[wid].at[j], bi2)
        ga0 = pltpu.async_copy(z_h.at[ai2.at[0]], ra0, sa0)
        gb0 = pltpu.async_copy(z_h.at[bi2.at[0]], rb0, sb0)
        ga1 = pltpu.async_copy(z_h.at[ai2.at[1]], ra1, sa1)
        gb1 = pltpu.async_copy(z_h.at[bi2.at[1]], rb1, sb1)
        ga0.wait()
        gb0.wait()
        out0 = pl.ds(wid * (NCH_D * CD) + j * (2 * CD), CD)
        pltpu.sync_copy(ra0, za_h.at[out0])
        pltpu.sync_copy(rb0, zb_h.at[out0])
        ga1.wait()
        gb1.wait()
        out1 = pl.ds(wid * (NCH_D * CD) + j * (2 * CD) + CD, CD)
        pltpu.sync_copy(ra1, za_h.at[out1])
        pltpu.sync_copy(rb1, zb_h.at[out1])


_decode_gather = pl.kernel(
    _decode_gather_body,
    out_type=jax.ShapeDtypeStruct((2, 2 * PP, D_H), jnp.float32),
    mesh=_sc_mesh,
    scratch_types=[
        pltpu.VMEM((2, CD), jnp.int32),
        pltpu.VMEM((2, CD), jnp.int32),
        pltpu.VMEM((CD, D_H), jnp.float32),
        pltpu.VMEM((CD, D_H), jnp.float32),
        pltpu.VMEM((CD, D_H), jnp.float32),
        pltpu.VMEM((CD, D_H), jnp.float32),
        pltpu.SemaphoreType.DMA,
        pltpu.SemaphoreType.DMA,
        pltpu.SemaphoreType.DMA,
        pltpu.SemaphoreType.DMA,
    ],
)


def kernel(x, edge_index, pos_edge_index, neg_edge_index, Wl1, bl1, Wr1, Wl2,
           bl2, Wr2):
    edges = edge_index.reshape(2, NW, NCH_E // 2, 2, CE)

    pad = PP - P
    pe = jnp.pad(pos_edge_index, ((0, 0), (0, pad)))
    ne = jnp.pad(neg_edge_index, ((0, 0), (0, pad)))
    ab = jnp.concatenate([pe, ne], axis=1).reshape(2, NW, NCH_D // 2, 2, CD)

    xW1 = _matmul(x, Wl1)
    cntp = _count_edges(edges)
    s1p = _segsum128(xW1, edges)
    xr1 = _matmul(x, Wr1)
    h, invc = _l1_post(s1p, cntp, xr1, bl1.reshape(1, D_H))
    s2p = _segsum128(h, edges)
    z = _l2_post(s2p, invc, h, Wl2, Wr2, bl2.reshape(1, D_OUT))
    zab = _decode_gather(z, ab)
    dots = _rowdot(zab[0], zab[1])[:, 0]
    pos_scores = dots[:P]
    neg_scores = dots[PP:PP + P]
    return (pos_scores, neg_scores)


# speedup vs baseline: 3.2875x; 1.1734x over previous
import jax
import jax.numpy as jnp
from jax import lax
from jax.experimental import pallas as pl
from jax.experimental.pallas import tpu as pltpu
from jax.experimental.pallas import tpu_sc as plsc

N = 10000
E = 320000
P = 100000
D_IN = 128
D_H = 128
D_OUT = 64

NC = 2
NS = 16
NW = NC * NS

NP = 10240
ROWS_PER_SUB = NP // NS
ZFILL = 40

NCH_E = 100
CE = 100

PP = 102400
NCH_D = 80
CD = 80

_HIGH = lax.Precision.HIGHEST

_sc_mesh = plsc.VectorSubcoreMesh(
    core_axis_name="c", subcore_axis_name="s", num_cores=NC, num_subcores=NS
)


def _dot(a, b):
    return lax.dot_general(a, b, (((1,), (0,)), ((), ())), precision=_HIGH)


def _mm_body(x_ref, w_ref, o_ref):
    o_ref[...] = _dot(x_ref[...], w_ref[...])


def _matmul(x, w, block_rows=2000):
    n, k = x.shape
    m = w.shape[1]
    return pl.pallas_call(
        _mm_body,
        grid=(n // block_rows,),
        in_specs=[
            pl.BlockSpec((block_rows, k), lambda i: (i, 0)),
            pl.BlockSpec((k, m), lambda i: (0, 0)),
        ],
        out_specs=pl.BlockSpec((block_rows, m), lambda i: (i, 0)),
        out_shape=jax.ShapeDtypeStruct((n, m), x.dtype),
    )(x, w)


def _l1_post_body(s1p_ref, cntp_ref, xr_ref, bl1_ref, h_ref, invc_ref):
    s = s1p_ref[0] + s1p_ref[1]
    cnt = cntp_ref[0, :, 0:1] + cntp_ref[1, :, 0:1]
    inv = 1.0 / jnp.maximum(cnt, 1.0)
    h_ref[...] = jnp.maximum(s * inv + bl1_ref[...] + xr_ref[...], 0.0)
    invc_ref[...] = jnp.broadcast_to(inv, invc_ref.shape)


def _l1_post(s1p, cntp, xr, bl1, block_rows=2000):
    return pl.pallas_call(
        _l1_post_body,
        grid=(N // block_rows,),
        in_specs=[
            pl.BlockSpec((2, block_rows, D_H), lambda i: (0, i, 0)),
            pl.BlockSpec((2, block_rows, D_H), lambda i: (0, i, 0)),
            pl.BlockSpec((block_rows, D_H), lambda i: (i, 0)),
            pl.BlockSpec((1, D_H), lambda i: (0, 0)),
        ],
        out_specs=[
            pl.BlockSpec((block_rows, D_H), lambda i: (i, 0)),
            pl.BlockSpec((block_rows, 16), lambda i: (i, 0)),
        ],
        out_shape=[
            jax.ShapeDtypeStruct((N, D_H), jnp.float32),
            jax.ShapeDtypeStruct((N, 16), jnp.float32),
        ],
    )(s1p, cntp, xr, bl1)


def _l2_post_body(s2p_ref, invc_ref, h_ref, wl2_ref, wr2_ref, bl2_ref, z_ref):
    mean = (s2p_ref[0] + s2p_ref[1]) * invc_ref[:, 0:1]
    z64 = _dot(mean, wl2_ref[...]) + bl2_ref[...] + _dot(
        h_ref[...], wr2_ref[...])
    z_ref[...] = jnp.concatenate(
        [z64, jnp.zeros_like(z64)], axis=1)


def _l2_post(s2p, invc, h, wl2, wr2, bl2, block_rows=2000):
    return pl.pallas_call(
        _l2_post_body,
        grid=(N // block_rows,),
        in_specs=[
            pl.BlockSpec((2, block_rows, D_H), lambda i: (0, i, 0)),
            pl.BlockSpec((block_rows, 16), lambda i: (i, 0)),
            pl.BlockSpec((block_rows, D_H), lambda i: (i, 0)),
            pl.BlockSpec((D_H, D_OUT), lambda i: (0, 0)),
            pl.BlockSpec((D_H, D_OUT), lambda i: (0, 0)),
            pl.BlockSpec((1, D_OUT), lambda i: (0, 0)),
        ],
        out_specs=pl.BlockSpec((block_rows, D_H), lambda i: (i, 0)),
        out_shape=jax.ShapeDtypeStruct((N, D_H), jnp.float32),
    )(s2p, invc, h, wl2, wr2, bl2)


def _rowdot_body(a_ref, b_ref, o_ref):
    o_ref[...] = jnp.sum(a_ref[...] * b_ref[...], axis=1, keepdims=True)


def _rowdot(a, b, block_rows=2048):
    n = a.shape[0]
    return pl.pallas_call(
        _rowdot_body,
        grid=(n // block_rows,),
        in_specs=[
            pl.BlockSpec((block_rows, D_H), lambda i: (i, 0)),
            pl.BlockSpec((block_rows, D_H), lambda i: (i, 0)),
        ],
        out_specs=pl.BlockSpec((block_rows, 1), lambda i: (i, 0)),
        out_shape=jax.ShapeDtypeStruct((n, 1), jnp.float32),
    )(a, b)


def _zero_vmem(ref, rows, width):
    zero = jnp.zeros((16,), jnp.float32)

    @pl.loop(0, rows)
    def _(r):
        @pl.loop(0, width, step=16)
        def _(c):
            ref[r, pl.ds(c, 16)] = zero


def _one_vmem(ref, rows, width):
    one = jnp.ones((16,), jnp.float32)

    @pl.loop(0, rows)
    def _(r):
        @pl.loop(0, width, step=16)
        def _(c):
            ref[r, pl.ds(c, 16)] = one


def _fill_stripe(acc, zbuf):
    sid = lax.axis_index("s")
    base = sid * ROWS_PER_SUB
    for k in range(ROWS_PER_SUB // ZFILL):
        pltpu.sync_copy(zbuf.at[pl.ds(0, ZFILL)],
                        acc.at[pl.ds(base + k * ZFILL, ZFILL)])


def _stage_table(tab_h, tab_sh):
    sid = lax.axis_index("s")

    @pl.when(sid < NS - 1)
    def _():
        sl = pl.ds(sid * ROWS_PER_SUB, ROWS_PER_SUB)
        pltpu.sync_copy(tab_h.at[sl], tab_sh.at[sl])

    @pl.when(sid == NS - 1)
    def _():
        last = (NS - 1) * ROWS_PER_SUB
        sl = pl.ds(last, N - last)
        pltpu.sync_copy(tab_h.at[sl], tab_sh.at[sl])


def _segsum128_body(table_h, edges_h, out_h, acc, src2, dst2, rows0, rows1,
                    sem0, sem1):
    src_h = edges_h.at[0]
    dst_h = edges_h.at[1]
    cid = lax.axis_index("c")
    sid = lax.axis_index("s")
    wid = cid * NS + sid

    _zero_vmem(rows0, ZFILL, D_H)
    _fill_stripe(acc, rows0)
    plsc.subcore_barrier()

    @pl.loop(0, NCH_E // 2)
    def _(j):
        pltpu.sync_copy(src_h.at[wid].at[j], src2)
        pltpu.sync_copy(dst_h.at[wid].at[j], dst2)
        g0 = pltpu.async_copy(table_h.at[src2.at[0]], rows0, sem0)
        g1 = pltpu.async_copy(table_h.at[src2.at[1]], rows1, sem1)
        g0.wait()
        pltpu.sync_copy(rows0, acc.at[dst2.at[0]], add=True)
        g1.wait()
        pltpu.sync_copy(rows1, acc.at[dst2.at[1]], add=True)

    plsc.subcore_barrier()
    stripe = pl.ds(sid * ROWS_PER_SUB, ROWS_PER_SUB)
    pltpu.sync_copy(acc.at[stripe], out_h.at[cid].at[stripe])


_segsum128 = pl.kernel(
    _segsum128_body,
    out_type=jax.ShapeDtypeStruct((NC, NP, D_H), jnp.float32),
    mesh=_sc_mesh,
    scratch_types=[
        pltpu.VMEM_SHARED((NP, D_H), jnp.float32),
        pltpu.VMEM((2, CE), jnp.int32),
        pltpu.VMEM((2, CE), jnp.int32),
        pltpu.VMEM((CE, D_H), jnp.float32),
        pltpu.VMEM((CE, D_H), jnp.float32),
        pltpu.SemaphoreType.DMA,
        pltpu.SemaphoreType.DMA,
    ],
)


def _count_body(edges_h, cnt_h, acc, dst2, ones_v):
    dst_h = edges_h.at[1]
    cid = lax.axis_index("c")
    sid = lax.axis_index("s")
    wid = cid * NS + sid

    _zero_vmem(ones_v, ZFILL, D_H)
    _fill_stripe(acc, ones_v)
    _one_vmem(ones_v, CE, D_H)
    plsc.subcore_barrier()

    @pl.loop(0, NCH_E // 2)
    def _(j):
        pltpu.sync_copy(dst_h.at[wid].at[j], dst2)
        pltpu.sync_copy(ones_v, acc.at[dst2.at[0]], add=True)
        pltpu.sync_copy(ones_v, acc.at[dst2.at[1]], add=True)

    plsc.subcore_barrier()
    stripe = pl.ds(sid * ROWS_PER_SUB, ROWS_PER_SUB)
    pltpu.sync_copy(acc.at[stripe], cnt_h.at[cid].at[stripe])


_count_edges = pl.kernel(
    _count_body,
    out_type=jax.ShapeDtypeStruct((NC, NP, D_H), jnp.float32),
    mesh=_sc_mesh,
    scratch_types=[
        pltpu.VMEM_SHARED((NP, D_H), jnp.float32),
        pltpu.VMEM((2, CE), jnp.int32),
        pltpu.VMEM((CE, D_H), jnp.float32),
    ],
)


def _decode_gather_body(z_h, ab_h, zab_h, ai2, bi2, ra0, rb0, ra1, rb1,
                        sa0, sb0, sa1, sb1):
    ai_h = ab_h.at[0]
    bi_h = ab_h.at[1]
    za_h = zab_h.at[0]
    zb_h = zab_h.at[1]
    cid = lax.axis_index("c")
    sid = lax.axis_index("s")
    wid = cid * NS + sid

    @pl.loop(0, NCH_D // 2)
    def _(j):
        pltpu.sync_copy(ai_h.at[wid].at[j], ai2)
        pltpu.sync_copy(bi_h.at[wid].at[j], bi2)
        ga0 = pltpu.async_copy(z_h.at[ai2.at[0]], ra0, sa0)
        gb0 = pltpu.async_copy(z_h.at[bi2.at[0]], rb0, sb0)
        ga1 = pltpu.async_copy(z_h.at[ai2.at[1]], ra1, sa1)
        gb1 = pltpu.async_copy(z_h.at[bi2.at[1]], rb1, sb1)
        ga0.wait()
        gb0.wait()
        out0 = pl.ds(wid * (NCH_D * CD) + j * (2 * CD), CD)
        oa0 = pltpu.async_copy(ra0, za_h.at[out0], sa0)
        ob0 = pltpu.async_copy(rb0, zb_h.at[out0], sb0)
        ga1.wait()
        gb1.wait()
        out1 = pl.ds(wid * (NCH_D * CD) + j * (2 * CD) + CD, CD)
        oa1 = pltpu.async_copy(ra1, za_h.at[out1], sa1)
        ob1 = pltpu.async_copy(rb1, zb_h.at[out1], sb1)
        oa0.wait()
        ob0.wait()
        oa1.wait()
        ob1.wait()


_decode_gather = pl.kernel(
    _decode_gather_body,
    out_type=jax.ShapeDtypeStruct((2, 2 * PP, D_H), jnp.float32),
    mesh=_sc_mesh,
    scratch_types=[
        pltpu.VMEM((2, CD), jnp.int32),
        pltpu.VMEM((2, CD), jnp.int32),
        pltpu.VMEM((CD, D_H), jnp.float32),
        pltpu.VMEM((CD, D_H), jnp.float32),
        pltpu.VMEM((CD, D_H), jnp.float32),
        pltpu.VMEM((CD, D_H), jnp.float32),
        pltpu.SemaphoreType.DMA,
        pltpu.SemaphoreType.DMA,
        pltpu.SemaphoreType.DMA,
        pltpu.SemaphoreType.DMA,
    ],
)


def kernel(x, edge_index, pos_edge_index, neg_edge_index, Wl1, bl1, Wr1, Wl2,
           bl2, Wr2):
    edges = edge_index.reshape(2, NW, NCH_E // 2, 2, CE)

    pad = PP - P
    pe = jnp.pad(pos_edge_index, ((0, 0), (0, pad)))
    ne = jnp.pad(neg_edge_index, ((0, 0), (0, pad)))
    ab = jnp.concatenate([pe, ne], axis=1).reshape(2, NW, NCH_D // 2, 2, CD)

    xW1 = _matmul(x, Wl1)
    cntp = _count_edges(edges)
    s1p = _segsum128(xW1, edges)
    xr1 = _matmul(x, Wr1)
    h, invc = _l1_post(s1p, cntp, xr1, bl1.reshape(1, D_H))
    s2p = _segsum128(h, edges)
    z = _l2_post(s2p, invc, h, Wl2, Wr2, bl2.reshape(1, D_OUT))
    zab = _decode_gather(z, ab)
    dots = _rowdot(zab[0], zab[1])[:, 0]
    pos_scores = dots[:P]
    neg_scores = dots[PP:PP + P]
    return (pos_scores, neg_scores)
